# baseline (speedup 1.0000x reference)
import functools

import jax
import jax.numpy as jnp
from jax.experimental import pallas as pl
from jax.experimental.pallas import tpu as pltpu

_C = 256
_NH = 16
_MM_DTYPE = jnp.bfloat16


def _rpn_body(x_ref, w_ref, cb_ref, hw_ref, hb_ref, oc_ref, ob_ref, xs_ref,
              *, S, BM, CH):
    IMG0 = S + 8
    SS = S * S

    xs_ref[0:IMG0, :] = jnp.zeros((IMG0, _C), _MM_DTYPE)
    xs_ref[IMG0 + SS:IMG0 + SS + S + 8, :] = jnp.zeros((S + 8, _C), _MM_DTYPE)
    for c in range(SS // CH):
        v = x_ref[0, :, c * CH:(c + 1) * CH]
        xs_ref[IMG0 + c * CH:IMG0 + (c + 1) * CH, :] = (
            jnp.transpose(v).astype(_MM_DTYPE))

    def band(b, _):
        m0 = b * BM
        xs3 = jnp.concatenate(
            [xs_ref[pl.ds(m0 + dy * S, BM + 16), :] for dy in range(3)],
            axis=1)
        accs = [jnp.dot(xs3, w_ref[dx], preferred_element_type=jnp.float32)
                for dx in range(3)]
        col = (jax.lax.broadcasted_iota(jnp.int32, (BM, 1), 0) + m0) & (S - 1)
        a0 = jnp.where(col != 0, accs[0][7:BM + 7], 0.0)
        a2 = jnp.where(col != S - 1, accs[2][9:BM + 9], 0.0)
        conv = a0 + accs[1][8:BM + 8] + a2
        t = jnp.maximum(conv + cb_ref[0, :][None, :], 0.0)
        h = jnp.dot(t.astype(_MM_DTYPE), hw_ref[...],
                    preferred_element_type=jnp.float32) + hb_ref[0, :][None, :]
        ht = jnp.transpose(h)
        oc_ref[0, :, pl.ds(m0, BM)] = ht[:3]
        ob_ref[0, :, pl.ds(m0, BM)] = ht[3:15]
        return 0

    jax.lax.fori_loop(0, SS // BM, band, 0, unroll=False)


@functools.partial(jax.jit, static_argnums=(5, 6, 7))
def _rpn_level(x, w3, cb, hw, hb, S, BM, CH):
    N = x.shape[0]
    SS = S * S
    xr = x.reshape(N, _C, SS)

    body = functools.partial(_rpn_body, S=S, BM=BM, CH=CH)
    o = pl.pallas_call(
        body,
        grid=(N,),
        in_specs=[
            pl.BlockSpec((1, _C, SS), lambda n: (n, 0, 0)),
            pl.BlockSpec((3, 3 * _C, _C), lambda n: (0, 0, 0)),
            pl.BlockSpec((1, _C), lambda n: (0, 0)),
            pl.BlockSpec((_C, _NH), lambda n: (0, 0)),
            pl.BlockSpec((1, _NH), lambda n: (0, 0)),
        ],
        out_specs=[pl.BlockSpec((1, 3, SS), lambda n: (n, 0, 0)),
                   pl.BlockSpec((1, 12, SS), lambda n: (n, 0, 0))],
        out_shape=[jax.ShapeDtypeStruct((N, 3, SS), jnp.float32),
                   jax.ShapeDtypeStruct((N, 12, SS), jnp.float32)],
        scratch_shapes=[pltpu.VMEM((SS + 2 * S + 16, _C), _MM_DTYPE)],
    )(xr, w3, cb, hw, hb)

    return o[0].reshape(N, 3, S, S), o[1].reshape(N, 12, S, S)


def kernel(feat_p3, feat_p4, feat_p5, feat_p6,
           conv_w, conv_b, cls_w, cls_b, bbox_w, bbox_b):
    w3 = jnp.transpose(conv_w, (3, 2, 1, 0)).reshape(3, 3 * _C, _C)
    w3 = w3.astype(_MM_DTYPE)
    cb = conv_b.reshape(1, _C)
    hw = jnp.concatenate([cls_w[:, :, 0, 0], bbox_w[:, :, 0, 0]], axis=0)
    hw = jnp.pad(hw, ((0, 1), (0, 0))).T.astype(_MM_DTYPE)
    hb = jnp.pad(jnp.concatenate([cls_b, bbox_b]), (0, 1)).reshape(1, _NH)

    cls_out, bbox_out = [], []
    for x, S, BM, CH in ((feat_p3, 128, 2048, 2048),
                         (feat_p4, 64, 2048, 2048),
                         (feat_p5, 32, 1024, 1024),
                         (feat_p6, 16, 256, 256)):
        c, b = _rpn_level(x, w3, cb, hw, hb, S, BM, CH)
        cls_out.append(c)
        bbox_out.append(b)
    return tuple(cls_out) + tuple(bbox_out)


# speedup vs baseline: 1.0075x; 1.0075x over previous
import jax
import jax.numpy as jnp
from jax.experimental import pallas as pl
from jax.experimental.pallas import tpu as pltpu

_C = 256
_NH = 16
_MM_DTYPE = jnp.bfloat16
_N = 2
_LEVELS = ((128, 2048), (64, 2048), (32, 1024), (16, 256))
def _pad(S):
    return ((S + 8 + 127) // 128) * 128


_STW = max(BM + 2 * _pad(S) for S, BM in _LEVELS)
_TR = max(BM + 2 * _pad(S) for S, BM in _LEVELS)


def _jobs():
    out = []
    for li, (S, BM) in enumerate(_LEVELS):
        SS = S * S
        for n in range(_N):
            for b in range(SS // BM):
                m0 = b * BM
                lo = max(0, m0 - _pad(S))
                hi = min(SS, m0 + BM + _pad(S))
                out.append((li, n, S, BM, m0, lo, hi))
    return out


def _rpn_body(x3_ref, x4_ref, x5_ref, x6_ref, w_ref, cb_ref, hw_ref, hb_ref,
              oc3, ob3, oc4, ob4, oc5, ob5, oc6, ob6,
              stage_ref, xt_ref, sem_ref):
    x_refs = (x3_ref, x4_ref, x5_ref, x6_ref)
    oc_refs = (oc3, oc4, oc5, oc6)
    ob_refs = (ob3, ob4, ob5, ob6)
    jobs = _jobs()

    def issue(i):
        li, n, S, BM, m0, lo, hi = jobs[i]
        buf = i % 2
        pltpu.make_async_copy(
            x_refs[li].at[n, :, lo:hi],
            stage_ref.at[buf, :, 0:hi - lo],
            sem_ref.at[buf]).start()

    issue(0)
    for i, (li, n, S, BM, m0, lo, hi) in enumerate(jobs):
        if i + 1 < len(jobs):
            issue(i + 1)
        buf = i % 2
        width = hi - lo
        pltpu.make_async_copy(
            x_refs[li].at[n, :, lo:hi],
            stage_ref.at[buf, :, 0:width],
            sem_ref.at[buf]).wait()

        PAD = _pad(S)
        ofs = lo - (m0 - PAD)
        rows = PAD + BM + S + 8
        if ofs:
            xt_ref[0:ofs, :] = jnp.zeros((ofs, _C), _MM_DTYPE)
        if ofs + width < rows:
            xt_ref[ofs + width:rows, :] = jnp.zeros(
                (rows - ofs - width, _C), _MM_DTYPE)
        xt_ref[ofs:ofs + width, :] = jnp.transpose(
            stage_ref[buf, :, 0:width]).astype(_MM_DTYPE)

        base = PAD - S - 8
        xs3 = jnp.concatenate(
            [xt_ref[base + dy * S:base + dy * S + BM + 16, :]
             for dy in range(3)],
            axis=1)
        accs = [jnp.dot(xs3, w_ref[dx], preferred_element_type=jnp.float32)
                for dx in range(3)]
        col = (jax.lax.broadcasted_iota(jnp.int32, (BM, 1), 0) + m0) & (S - 1)
        a0 = jnp.where(col != 0, accs[0][7:BM + 7], 0.0)
        a2 = jnp.where(col != S - 1, accs[2][9:BM + 9], 0.0)
        conv = a0 + accs[1][8:BM + 8] + a2
        t = jnp.maximum(conv + cb_ref[0, :][None, :], 0.0)
        h = jnp.dot(t.astype(_MM_DTYPE), hw_ref[...],
                    preferred_element_type=jnp.float32) + hb_ref[0, :][None, :]
        ht = jnp.transpose(h)
        oc_refs[li][n, :, m0:m0 + BM] = ht[:3]
        ob_refs[li][n, :, m0:m0 + BM] = ht[3:15]


@jax.jit
def _rpn_all(x3, x4, x5, x6, w3, cb, hw, hb):
    feats = (x3, x4, x5, x6)
    xr = [x.reshape(_N, _C, -1) for x in feats]
    out_shape = []
    for S, _ in _LEVELS:
        out_shape += [jax.ShapeDtypeStruct((_N, 3, S * S), jnp.float32),
                      jax.ShapeDtypeStruct((_N, 12, S * S), jnp.float32)]

    any_spec = pl.BlockSpec(memory_space=pl.ANY)
    o = pl.pallas_call(
        _rpn_body,
        in_specs=[any_spec] * 4 + [
            pl.BlockSpec((3, 3 * _C, _C), lambda: (0, 0, 0)),
            pl.BlockSpec((1, _C), lambda: (0, 0)),
            pl.BlockSpec((_C, _NH), lambda: (0, 0)),
            pl.BlockSpec((1, _NH), lambda: (0, 0)),
        ],
        out_shape=out_shape,
        scratch_shapes=[
            pltpu.VMEM((2, _C, _STW), jnp.float32),
            pltpu.VMEM((_TR, _C), _MM_DTYPE),
            pltpu.SemaphoreType.DMA((2,)),
        ],
    )(*xr, w3, cb, hw, hb)

    cls_out, bbox_out = [], []
    for li, (S, _) in enumerate(_LEVELS):
        cls_out.append(o[2 * li].reshape(_N, 3, S, S))
        bbox_out.append(o[2 * li + 1].reshape(_N, 12, S, S))
    return tuple(cls_out) + tuple(bbox_out)


def kernel(feat_p3, feat_p4, feat_p5, feat_p6,
           conv_w, conv_b, cls_w, cls_b, bbox_w, bbox_b):
    w3 = jnp.transpose(conv_w, (3, 2, 1, 0)).reshape(3, 3 * _C, _C)
    w3 = w3.astype(_MM_DTYPE)
    cb = conv_b.reshape(1, _C)
    hw = jnp.concatenate([cls_w[:, :, 0, 0], bbox_w[:, :, 0, 0]], axis=0)
    hw = jnp.pad(hw, ((0, 1), (0, 0))).T.astype(_MM_DTYPE)
    hb = jnp.pad(jnp.concatenate([cls_b, bbox_b]), (0, 1)).reshape(1, _NH)

    return _rpn_all(feat_p3, feat_p4, feat_p5, feat_p6, w3, cb, hw, hb)
